# baseline (speedup 1.0000x reference)
import functools

import jax
import jax.numpy as jnp
from jax import lax
from jax.experimental import pallas as pl
from jax.experimental.pallas import tpu as pltpu
from jax.experimental.pallas import tpu_sc as plsc

_E = 8
_K = 2
_T = 2048
_D = 1024
_F = 512
_BLK = 256
_NT = (_T * _K) // _BLK + _E - 1
_NPAD = 6144
_NC, _NS = 2, 16
_NW = _NC * _NS
_GCH = 32
_CCH = 32


def _cumsum0(a, n):
    k = 1
    while k < n:
        a = a + jnp.concatenate(
            [jnp.zeros((k, a.shape[1]), a.dtype), a[:-k]], axis=0)
        k *= 2
    return a


def _bf16_bits(xf):
    u = lax.bitcast_convert_type(xf, jnp.int32)
    return u + 0x7FFF + jnp.bitwise_and(lax.shift_right_logical(u, 16), 1)


def _gate_body(x_ref, wg_ref, dst1_ref, dst2_ref, s1_ref, s2_ref, cnt_ref,
               xb_ref):
    x = x_ref[...]
    rl = _bf16_bits(x[:, :_D // 2])
    rh = _bf16_bits(x[:, _D // 2:])
    xb_ref[...] = jnp.bitwise_or(
        jnp.bitwise_and(rh, jnp.int32(-65536)),
        jnp.bitwise_and(lax.shift_right_logical(rl, 16), 0xFFFF))
    logits = lax.dot_general(x, wg_ref[...], (((1,), (1,)), ((), ())),
                             preferred_element_type=jnp.float32)
    cols = lax.broadcasted_iota(jnp.int32, logits.shape, 1)
    m1 = jnp.max(logits, axis=1, keepdims=True)
    i1 = jnp.min(jnp.where(logits == m1, cols, _E), axis=1, keepdims=True)
    oh1 = (cols == i1).astype(jnp.float32)
    rest = jnp.where(cols == i1, -jnp.inf, logits)
    m2 = jnp.max(rest, axis=1, keepdims=True)
    i2 = jnp.min(jnp.where(rest == m2, cols, _E), axis=1, keepdims=True)
    oh2 = (cols == i2).astype(jnp.float32)
    s1 = 1.0 / (1.0 + jnp.exp(m2 - m1))

    c0 = _cumsum0(oh1, _T)
    tot0 = c0[_T - 1:_T, :]
    c1 = _cumsum0(oh2, _T)
    tot1 = c1[_T - 1:_T, :]
    rank1 = jnp.sum((c0 - oh1) * oh1, axis=1, keepdims=True)
    rank2 = jnp.sum((c1 - oh2 + tot0) * oh2, axis=1, keepdims=True)
    cnt = tot0 + tot1
    cap = jnp.floor((cnt + (_BLK - 1)) / _BLK) * _BLK
    ei = lax.broadcasted_iota(jnp.int32, (_E, _E), 0)
    ej = lax.broadcasted_iota(jnp.int32, (_E, _E), 1)
    ltm = (ei < ej).astype(jnp.float32)
    off = lax.dot_general(cap, ltm, (((1,), (0,)), ((), ())),
                          preferred_element_type=jnp.float32)
    dst1_ref[...] = (jnp.sum(off * oh1, axis=1, keepdims=True)
                     + rank1).astype(jnp.int32)
    dst2_ref[...] = (jnp.sum(off * oh2, axis=1, keepdims=True)
                     + rank2).astype(jnp.int32)
    s1_ref[...] = s1
    s2_ref[...] = 1.0 - s1
    cnt_ref[...] = cnt.astype(jnp.int32)


def _gate_call(x, w_gate):
    return pl.pallas_call(
        _gate_body,
        out_shape=(
            jax.ShapeDtypeStruct((_T, 1), jnp.int32),
            jax.ShapeDtypeStruct((_T, 1), jnp.int32),
            jax.ShapeDtypeStruct((_T, 1), jnp.float32),
            jax.ShapeDtypeStruct((_T, 1), jnp.float32),
            jax.ShapeDtypeStruct((1, _E), jnp.int32),
            jax.ShapeDtypeStruct((_T, _D // 2), jnp.int32),
        ),
    )(x, w_gate)


def _mesh():
    return plsc.VectorSubcoreMesh(core_axis_name="c", subcore_axis_name="s",
                                  num_cores=_NC, num_subcores=_NS)


@functools.cache
def _dispatch_call():
    return functools.partial(
        pl.kernel,
        out_type=(jax.ShapeDtypeStruct((_NPAD, _D // 2), jnp.int32),
                  jax.ShapeDtypeStruct((_NPAD,), jnp.float32)),
        mesh=_mesh(),
        scratch_types=[
            pltpu.VMEM((_NPAD,), jnp.int32),
            pltpu.VMEM((_NPAD,), jnp.float32),
            pltpu.VMEM((_T,), jnp.int32),
            pltpu.VMEM((_T,), jnp.int32),
            pltpu.VMEM((_T,), jnp.float32),
            pltpu.VMEM((_T,), jnp.float32),
            pltpu.VMEM((_GCH, _D // 2), jnp.int32),
            pltpu.VMEM((_GCH, _D // 2), jnp.int32),
            pltpu.VMEM((_NPAD // _NW,), jnp.int32),
            pltpu.SemaphoreType.DMA,
            pltpu.SemaphoreType.DMA,
        ],
        compiler_params=pltpu.CompilerParams(needs_layout_passes=False),
    )(_dispatch)


def _dispatch(xb_hbm, dst1_hbm, dst2_hbm, s1_hbm, s2_hbm, xs_hbm, ws_hbm,
              tok_loc, w_loc, dst1_loc, dst2_loc, s1_loc, s2_loc,
              rows0, rows1, idxbuf, sem0, sem1):
    wid = lax.axis_index("s") * _NC + lax.axis_index("c")
    pltpu.sync_copy(dst1_hbm, dst1_loc)
    pltpu.sync_copy(dst2_hbm, dst2_loc)
    pltpu.sync_copy(s1_hbm, s1_loc)
    pltpu.sync_copy(s2_hbm, s2_loc)

    def _zero(i, _):
        tok_loc[pl.ds(i * 16, 16)] = jnp.zeros((16,), jnp.int32)
        w_loc[pl.ds(i * 16, 16)] = jnp.zeros((16,), jnp.float32)
        return ()
    lax.fori_loop(0, _NPAD // 16, _zero, (), unroll=4)

    def _scatter1(i, _):
        sl = pl.ds(i * 16, 16)
        tv = lax.iota(jnp.int32, 16) + i * 16
        plsc.store_scatter(tok_loc, [dst1_loc[sl]], tv)
        plsc.store_scatter(w_loc, [dst1_loc[sl]], s1_loc[sl])
        return ()
    lax.fori_loop(0, _T // 16, _scatter1, (), unroll=4)

    def _scatter2(i, _):
        sl = pl.ds(i * 16, 16)
        tv = lax.iota(jnp.int32, 16) + i * 16
        plsc.store_scatter(tok_loc, [dst2_loc[sl]], tv)
        plsc.store_scatter(w_loc, [dst2_loc[sl]], s2_loc[sl])
        return ()
    lax.fori_loop(0, _T // 16, _scatter2, (), unroll=4)

    span = _NPAD // _NW
    base = wid * span
    pltpu.sync_copy(w_loc.at[pl.ds(base, span)], ws_hbm.at[pl.ds(base, span)])

    def _cpidx(i, _):
        idxbuf[pl.ds(i * 16, 16)] = tok_loc[pl.ds(base + i * 16, 16)]
        return ()
    lax.fori_loop(0, span // 16, _cpidx, (), unroll=4)

    nch = span // _GCH
    rows = (rows0, rows1)
    sems = (sem0, sem1)
    cps = [None, None]
    cps[0] = pltpu.async_copy(
        xb_hbm.at[idxbuf.at[pl.ds(0, _GCH)]], rows0, sem0)
    for c in range(nch):
        if c + 1 < nch:
            cps[(c + 1) % 2] = pltpu.async_copy(
                xb_hbm.at[idxbuf.at[pl.ds((c + 1) * _GCH, _GCH)]],
                rows[(c + 1) % 2], sems[(c + 1) % 2])
        cps[c % 2].wait()
        pltpu.sync_copy(rows[c % 2], xs_hbm.at[pl.ds(base + c * _GCH, _GCH)])


def _ffn_body(sched_ref, xs_ref, w1_ref, w2_ref, w3_ref, ws_ref, ys_ref):
    j = pl.program_id(0)

    @pl.when(sched_ref[2, j] == 1)
    def _():
        w = xs_ref[...]
        xlo = lax.bitcast_convert_type(
            lax.shift_left(w, 16), jnp.float32).astype(jnp.bfloat16)
        xhi = lax.bitcast_convert_type(
            jnp.bitwise_and(w, jnp.int32(-65536)),
            jnp.float32).astype(jnp.bfloat16)
        w1 = w1_ref[0].astype(jnp.bfloat16)
        w3 = w3_ref[0].astype(jnp.bfloat16)
        w2 = w2_ref[0].astype(jnp.bfloat16)
        dn = (((1,), (1,)), ((), ()))
        a = (lax.dot_general(xlo, w1[:, :_D // 2], dn,
                             preferred_element_type=jnp.float32)
             + lax.dot_general(xhi, w1[:, _D // 2:], dn,
                               preferred_element_type=jnp.float32))
        b = (lax.dot_general(xlo, w3[:, :_D // 2], dn,
                             preferred_element_type=jnp.float32)
             + lax.dot_general(xhi, w3[:, _D // 2:], dn,
                               preferred_element_type=jnp.float32))
        h = (a * lax.logistic(a) * b).astype(jnp.bfloat16)
        oe = lax.dot_general(h, w2, (((1,), (0,)), ((), ())),
                             preferred_element_type=jnp.float32)
        ys_ref[...] = oe * ws_ref[...]


def _ffn_call(sched, xs, w1, w2, w3, ws):
    grid_spec = pltpu.PrefetchScalarGridSpec(
        num_scalar_prefetch=1,
        grid=(_NT,),
        in_specs=[
            pl.BlockSpec((_BLK, _D // 2), lambda j, s: (s[0, j], 0)),
            pl.BlockSpec((1, _F, _D), lambda j, s: (s[1, j], 0, 0)),
            pl.BlockSpec((1, _F, _D), lambda j, s: (s[1, j], 0, 0)),
            pl.BlockSpec((1, _F, _D), lambda j, s: (s[1, j], 0, 0)),
            pl.BlockSpec((_BLK, 1), lambda j, s: (s[0, j], 0)),
        ],
        out_specs=pl.BlockSpec((_BLK, _D), lambda j, s: (s[0, j], 0)),
    )
    return pl.pallas_call(
        _ffn_body,
        grid_spec=grid_spec,
        out_shape=jax.ShapeDtypeStruct((_NPAD, _D), jnp.float32),
        compiler_params=pltpu.CompilerParams(
            dimension_semantics=("arbitrary",)),
    )(sched, xs, w1, w2, w3, ws)


@functools.cache
def _combine_call():
    return functools.partial(
        pl.kernel,
        out_type=jax.ShapeDtypeStruct((_T, _D), jnp.float32),
        mesh=_mesh(),
        scratch_types=[
            pltpu.VMEM((_T // _NW,), jnp.int32),
            pltpu.VMEM((_T // _NW,), jnp.int32),
            pltpu.VMEM((_CCH, _D), jnp.float32),
            pltpu.VMEM((_CCH, _D), jnp.float32),
            pltpu.SemaphoreType.DMA,
            pltpu.SemaphoreType.DMA,
        ],
        compiler_params=pltpu.CompilerParams(needs_layout_passes=False),
    )(_combine)


def _combine(ys_hbm, dst1_hbm, dst2_hbm, out_hbm,
             idx1, idx2, buf1, buf2, sem1, sem2):
    wid = lax.axis_index("s") * _NC + lax.axis_index("c")
    span = _T // _NW
    base = wid * span
    pltpu.sync_copy(dst1_hbm.at[pl.ds(base, span)], idx1)
    pltpu.sync_copy(dst2_hbm.at[pl.ds(base, span)], idx2)
    for c in range(span // _CCH):
        a1 = pltpu.async_copy(ys_hbm.at[idx1.at[pl.ds(c * _CCH, _CCH)]],
                              buf1, sem1)
        a2 = pltpu.async_copy(ys_hbm.at[idx2.at[pl.ds(c * _CCH, _CCH)]],
                              buf2, sem2)
        a1.wait()
        a2.wait()

        def _add(r, _):
            for cc in range(_D // 16):
                sl = pl.ds(cc * 16, 16)
                buf1[r, sl] = buf1[r, sl] + buf2[r, sl]
            return ()
        lax.fori_loop(0, _CCH, _add, ())
        pltpu.sync_copy(buf1, out_hbm.at[pl.ds(base + c * _CCH, _CCH)])


def kernel(x, w_gate, w1, w2, w3):
    dst1, dst2, s1, s2, cnt, xb_i = _gate_call(x, w_gate)
    d1 = dst1.reshape(_T)
    d2 = dst2.reshape(_T)
    cnt = cnt.reshape(_E)
    tiles = (cnt + _BLK - 1) // _BLK
    at = jnp.sum(tiles)
    jr = jnp.arange(_NT, dtype=jnp.int32)
    jcl = jnp.minimum(jr, at - 1)
    exps = jnp.searchsorted(jnp.cumsum(tiles), jcl, side="right")
    sched = jnp.stack([jcl, exps.astype(jnp.int32),
                       (jr < at).astype(jnp.int32)])
    xs_i, ws = _dispatch_call()(xb_i, d1, d2, s1.reshape(_T), s2.reshape(_T))
    ys = _ffn_call(sched, xs_i, w1, w2, w3, ws.reshape(_NPAD, 1))
    return _combine_call()(ys, d1, d2)


# speedup vs baseline: 1.0018x; 1.0018x over previous
import functools

import jax
import jax.numpy as jnp
from jax import lax
from jax.experimental import pallas as pl
from jax.experimental.pallas import tpu as pltpu
from jax.experimental.pallas import tpu_sc as plsc

_E = 8
_K = 2
_T = 2048
_D = 1024
_F = 512
_BLK = 256
_NT = (_T * _K) // _BLK + _E - 1
_NPAD = 6144
_NC, _NS = 2, 16
_NW = _NC * _NS
_GCH = 32
_CCH = 32


def _cumsum0(a, n):
    k = 1
    while k < n:
        a = a + jnp.concatenate(
            [jnp.zeros((k, a.shape[1]), a.dtype), a[:-k]], axis=0)
        k *= 2
    return a


def _bf16_bits(xf):
    u = lax.bitcast_convert_type(xf, jnp.int32)
    return u + 0x7FFF + jnp.bitwise_and(lax.shift_right_logical(u, 16), 1)


def _gate_body(x_ref, wg_ref, dst1_ref, dst2_ref, s1_ref, s2_ref, cnt_ref,
               xb_ref):
    x = x_ref[...]
    rl = _bf16_bits(x[:, :_D // 2])
    rh = _bf16_bits(x[:, _D // 2:])
    xb_ref[...] = jnp.bitwise_or(
        jnp.bitwise_and(rh, jnp.int32(-65536)),
        jnp.bitwise_and(lax.shift_right_logical(rl, 16), 0xFFFF))
    logits = lax.dot_general(x, wg_ref[...], (((1,), (1,)), ((), ())),
                             preferred_element_type=jnp.float32)
    cols = lax.broadcasted_iota(jnp.int32, logits.shape, 1)
    m1 = jnp.max(logits, axis=1, keepdims=True)
    i1 = jnp.min(jnp.where(logits == m1, cols, _E), axis=1, keepdims=True)
    oh1 = (cols == i1).astype(jnp.float32)
    rest = jnp.where(cols == i1, -jnp.inf, logits)
    m2 = jnp.max(rest, axis=1, keepdims=True)
    i2 = jnp.min(jnp.where(rest == m2, cols, _E), axis=1, keepdims=True)
    oh2 = (cols == i2).astype(jnp.float32)
    s1 = 1.0 / (1.0 + jnp.exp(m2 - m1))

    c0 = _cumsum0(oh1, _T)
    tot0 = c0[_T - 1:_T, :]
    c1 = _cumsum0(oh2, _T)
    tot1 = c1[_T - 1:_T, :]
    rank1 = jnp.sum((c0 - oh1) * oh1, axis=1, keepdims=True)
    rank2 = jnp.sum((c1 - oh2 + tot0) * oh2, axis=1, keepdims=True)
    cnt = tot0 + tot1
    cap = jnp.floor((cnt + (_BLK - 1)) / _BLK) * _BLK
    ei = lax.broadcasted_iota(jnp.int32, (_E, _E), 0)
    ej = lax.broadcasted_iota(jnp.int32, (_E, _E), 1)
    ltm = (ei < ej).astype(jnp.float32)
    off = lax.dot_general(cap, ltm, (((1,), (0,)), ((), ())),
                          preferred_element_type=jnp.float32)
    dst1_ref[...] = (jnp.sum(off * oh1, axis=1, keepdims=True)
                     + rank1).astype(jnp.int32)
    dst2_ref[...] = (jnp.sum(off * oh2, axis=1, keepdims=True)
                     + rank2).astype(jnp.int32)
    s1_ref[...] = s1
    s2_ref[...] = 1.0 - s1
    cnt_ref[...] = cnt.astype(jnp.int32)


def _gate_call(x, w_gate):
    return pl.pallas_call(
        _gate_body,
        out_shape=(
            jax.ShapeDtypeStruct((_T, 1), jnp.int32),
            jax.ShapeDtypeStruct((_T, 1), jnp.int32),
            jax.ShapeDtypeStruct((_T, 1), jnp.float32),
            jax.ShapeDtypeStruct((_T, 1), jnp.float32),
            jax.ShapeDtypeStruct((1, _E), jnp.int32),
            jax.ShapeDtypeStruct((_T, _D // 2), jnp.int32),
        ),
    )(x, w_gate)


def _mesh():
    return plsc.VectorSubcoreMesh(core_axis_name="c", subcore_axis_name="s",
                                  num_cores=_NC, num_subcores=_NS)


@functools.cache
def _dispatch_call():
    return functools.partial(
        pl.kernel,
        out_type=(jax.ShapeDtypeStruct((_NPAD, _D // 2), jnp.int32),
                  jax.ShapeDtypeStruct((_NPAD,), jnp.float32)),
        mesh=_mesh(),
        scratch_types=[
            pltpu.VMEM((_NPAD,), jnp.int32),
            pltpu.VMEM((_NPAD,), jnp.float32),
            pltpu.VMEM((_T,), jnp.int32),
            pltpu.VMEM((_T,), jnp.int32),
            pltpu.VMEM((_T,), jnp.float32),
            pltpu.VMEM((_T,), jnp.float32),
            pltpu.VMEM((_GCH, _D // 2), jnp.int32),
            pltpu.VMEM((_GCH, _D // 2), jnp.int32),
            pltpu.VMEM((_NPAD // _NW,), jnp.int32),
            pltpu.SemaphoreType.DMA,
            pltpu.SemaphoreType.DMA,
        ],
        compiler_params=pltpu.CompilerParams(needs_layout_passes=False),
    )(_dispatch)


def _dispatch(xb_hbm, dst1_hbm, dst2_hbm, s1_hbm, s2_hbm, xs_hbm, ws_hbm,
              tok_loc, w_loc, dst1_loc, dst2_loc, s1_loc, s2_loc,
              rows0, rows1, idxbuf, sem0, sem1):
    wid = lax.axis_index("s") * _NC + lax.axis_index("c")
    with jax.named_scope("disp_stage"):
        pltpu.sync_copy(dst1_hbm, dst1_loc)
        pltpu.sync_copy(dst2_hbm, dst2_loc)
        pltpu.sync_copy(s1_hbm, s1_loc)
        pltpu.sync_copy(s2_hbm, s2_loc)

    def _zero(i, _):
        tok_loc[pl.ds(i * 16, 16)] = jnp.zeros((16,), jnp.int32)
        w_loc[pl.ds(i * 16, 16)] = jnp.zeros((16,), jnp.float32)
        return ()
    with jax.named_scope("disp_zero"):
        lax.fori_loop(0, _NPAD // 16, _zero, (), unroll=4)

    def _scatter1(i, _):
        sl = pl.ds(i * 16, 16)
        tv = lax.iota(jnp.int32, 16) + i * 16
        plsc.store_scatter(tok_loc, [dst1_loc[sl]], tv)
        plsc.store_scatter(w_loc, [dst1_loc[sl]], s1_loc[sl])
        return ()
    with jax.named_scope("disp_scat1"):
        lax.fori_loop(0, _T // 16, _scatter1, (), unroll=4)

    def _scatter2(i, _):
        sl = pl.ds(i * 16, 16)
        tv = lax.iota(jnp.int32, 16) + i * 16
        plsc.store_scatter(tok_loc, [dst2_loc[sl]], tv)
        plsc.store_scatter(w_loc, [dst2_loc[sl]], s2_loc[sl])
        return ()
    with jax.named_scope("disp_scat2"):
        lax.fori_loop(0, _T // 16, _scatter2, (), unroll=4)

    span = _NPAD // _NW
    base = wid * span
    with jax.named_scope("disp_ws"):
        pltpu.sync_copy(w_loc.at[pl.ds(base, span)],
                        ws_hbm.at[pl.ds(base, span)])

    def _cpidx(i, _):
        idxbuf[pl.ds(i * 16, 16)] = tok_loc[pl.ds(base + i * 16, 16)]
        return ()
    with jax.named_scope("disp_cpidx"):
        lax.fori_loop(0, span // 16, _cpidx, (), unroll=4)

    with jax.named_scope("disp_gather"):
        nch = span // _GCH
        rows = (rows0, rows1)
        sems = (sem0, sem1)
        cps = [None, None]
        cps[0] = pltpu.async_copy(
            xb_hbm.at[idxbuf.at[pl.ds(0, _GCH)]], rows0, sem0)
        for c in range(nch):
            if c + 1 < nch:
                cps[(c + 1) % 2] = pltpu.async_copy(
                    xb_hbm.at[idxbuf.at[pl.ds((c + 1) * _GCH, _GCH)]],
                    rows[(c + 1) % 2], sems[(c + 1) % 2])
            cps[c % 2].wait()
            pltpu.sync_copy(rows[c % 2],
                            xs_hbm.at[pl.ds(base + c * _GCH, _GCH)])


def _ffn_body(sched_ref, xs_ref, w1_ref, w2_ref, w3_ref, ws_ref, ys_ref):
    j = pl.program_id(0)

    @pl.when(sched_ref[2, j] == 1)
    def _():
        w = xs_ref[...]
        xlo = lax.bitcast_convert_type(
            lax.shift_left(w, 16), jnp.float32).astype(jnp.bfloat16)
        xhi = lax.bitcast_convert_type(
            jnp.bitwise_and(w, jnp.int32(-65536)),
            jnp.float32).astype(jnp.bfloat16)
        w1 = w1_ref[0].astype(jnp.bfloat16)
        w3 = w3_ref[0].astype(jnp.bfloat16)
        w2 = w2_ref[0].astype(jnp.bfloat16)
        dn = (((1,), (1,)), ((), ()))
        a = (lax.dot_general(xlo, w1[:, :_D // 2], dn,
                             preferred_element_type=jnp.float32)
             + lax.dot_general(xhi, w1[:, _D // 2:], dn,
                               preferred_element_type=jnp.float32))
        b = (lax.dot_general(xlo, w3[:, :_D // 2], dn,
                             preferred_element_type=jnp.float32)
             + lax.dot_general(xhi, w3[:, _D // 2:], dn,
                               preferred_element_type=jnp.float32))
        h = (a * lax.logistic(a) * b).astype(jnp.bfloat16)
        oe = lax.dot_general(h, w2, (((1,), (0,)), ((), ())),
                             preferred_element_type=jnp.float32)
        ys_ref[...] = oe * ws_ref[...]


def _ffn_call(sched, xs, w1, w2, w3, ws):
    grid_spec = pltpu.PrefetchScalarGridSpec(
        num_scalar_prefetch=1,
        grid=(_NT,),
        in_specs=[
            pl.BlockSpec((_BLK, _D // 2), lambda j, s: (s[0, j], 0)),
            pl.BlockSpec((1, _F, _D), lambda j, s: (s[1, j], 0, 0)),
            pl.BlockSpec((1, _F, _D), lambda j, s: (s[1, j], 0, 0)),
            pl.BlockSpec((1, _F, _D), lambda j, s: (s[1, j], 0, 0)),
            pl.BlockSpec((_BLK, 1), lambda j, s: (s[0, j], 0)),
        ],
        out_specs=pl.BlockSpec((_BLK, _D), lambda j, s: (s[0, j], 0)),
    )
    return pl.pallas_call(
        _ffn_body,
        grid_spec=grid_spec,
        out_shape=jax.ShapeDtypeStruct((_NPAD, _D), jnp.float32),
        compiler_params=pltpu.CompilerParams(
            dimension_semantics=("arbitrary",)),
    )(sched, xs, w1, w2, w3, ws)


@functools.cache
def _combine_call():
    return functools.partial(
        pl.kernel,
        out_type=jax.ShapeDtypeStruct((_T, _D), jnp.float32),
        mesh=_mesh(),
        scratch_types=[
            pltpu.VMEM((_T // _NW,), jnp.int32),
            pltpu.VMEM((_T // _NW,), jnp.int32),
            pltpu.VMEM((_CCH, _D), jnp.float32),
            pltpu.VMEM((_CCH, _D), jnp.float32),
            pltpu.SemaphoreType.DMA,
            pltpu.SemaphoreType.DMA,
        ],
        compiler_params=pltpu.CompilerParams(needs_layout_passes=False),
    )(_combine)


def _combine(ys_hbm, dst1_hbm, dst2_hbm, out_hbm,
             idx1, idx2, buf1, buf2, sem1, sem2):
    wid = lax.axis_index("s") * _NC + lax.axis_index("c")
    span = _T // _NW
    base = wid * span
    pltpu.sync_copy(dst1_hbm.at[pl.ds(base, span)], idx1)
    pltpu.sync_copy(dst2_hbm.at[pl.ds(base, span)], idx2)
    for c in range(span // _CCH):
        a1 = pltpu.async_copy(ys_hbm.at[idx1.at[pl.ds(c * _CCH, _CCH)]],
                              buf1, sem1)
        a2 = pltpu.async_copy(ys_hbm.at[idx2.at[pl.ds(c * _CCH, _CCH)]],
                              buf2, sem2)
        a1.wait()
        a2.wait()

        def _add(r, _):
            for cc in range(_D // 16):
                sl = pl.ds(cc * 16, 16)
                buf1[r, sl] = buf1[r, sl] + buf2[r, sl]
            return ()
        lax.fori_loop(0, _CCH, _add, ())
        pltpu.sync_copy(buf1, out_hbm.at[pl.ds(base + c * _CCH, _CCH)])


def kernel(x, w_gate, w1, w2, w3):
    dst1, dst2, s1, s2, cnt, xb_i = _gate_call(x, w_gate)
    d1 = dst1.reshape(_T)
    d2 = dst2.reshape(_T)
    cnt = cnt.reshape(_E)
    tiles = (cnt + _BLK - 1) // _BLK
    at = jnp.sum(tiles)
    jr = jnp.arange(_NT, dtype=jnp.int32)
    jcl = jnp.minimum(jr, at - 1)
    exps = jnp.searchsorted(jnp.cumsum(tiles), jcl, side="right")
    sched = jnp.stack([jcl, exps.astype(jnp.int32),
                       (jr < at).astype(jnp.int32)])
    xs_i, ws = _dispatch_call()(xb_i, d1, d2, s1.reshape(_T), s2.reshape(_T))
    ys = _ffn_call(sched, xs_i, w1, w2, w3, ws.reshape(_NPAD, 1))
    return _combine_call()(ys, d1, d2)


# speedup vs baseline: 1.7297x; 1.7266x over previous
import functools

import jax
import jax.numpy as jnp
from jax import lax
from jax.experimental import pallas as pl
from jax.experimental.pallas import tpu as pltpu
from jax.experimental.pallas import tpu_sc as plsc

_E = 8
_K = 2
_T = 2048
_D = 1024
_F = 512
_BLK = 256
_NT = (_T * _K) // _BLK + _E - 1
_NPAD = 6144
_NC, _NS = 2, 16
_NW = _NC * _NS
_GCH = 32
_CCH = 32


def _cumsum0(a, n):
    k = 1
    while k < n:
        a = a + jnp.concatenate(
            [jnp.zeros((k, a.shape[1]), a.dtype), a[:-k]], axis=0)
        k *= 2
    return a


def _bf16_bits(xf):
    u = lax.bitcast_convert_type(xf, jnp.int32)
    return u + 0x7FFF + jnp.bitwise_and(lax.shift_right_logical(u, 16), 1)


def _gate_body(x_ref, wg_ref, dst1_ref, dst2_ref, s1_ref, s2_ref, cnt_ref,
               xb_ref):
    x = x_ref[...]
    rl = _bf16_bits(x[:, :_D // 2])
    rh = _bf16_bits(x[:, _D // 2:])
    xb_ref[...] = jnp.bitwise_or(
        jnp.bitwise_and(rh, jnp.int32(-65536)),
        jnp.bitwise_and(lax.shift_right_logical(rl, 16), 0xFFFF))
    logits = lax.dot_general(x, wg_ref[...], (((1,), (1,)), ((), ())),
                             preferred_element_type=jnp.float32)
    cols = lax.broadcasted_iota(jnp.int32, logits.shape, 1)
    m1 = jnp.max(logits, axis=1, keepdims=True)
    i1 = jnp.min(jnp.where(logits == m1, cols, _E), axis=1, keepdims=True)
    oh1 = (cols == i1).astype(jnp.float32)
    rest = jnp.where(cols == i1, -jnp.inf, logits)
    m2 = jnp.max(rest, axis=1, keepdims=True)
    i2 = jnp.min(jnp.where(rest == m2, cols, _E), axis=1, keepdims=True)
    oh2 = (cols == i2).astype(jnp.float32)
    s1 = 1.0 / (1.0 + jnp.exp(m2 - m1))

    c0 = _cumsum0(oh1, _T)
    tot0 = c0[_T - 1:_T, :]
    c1 = _cumsum0(oh2, _T)
    tot1 = c1[_T - 1:_T, :]
    rank1 = jnp.sum((c0 - oh1) * oh1, axis=1, keepdims=True)
    rank2 = jnp.sum((c1 - oh2 + tot0) * oh2, axis=1, keepdims=True)
    cnt = tot0 + tot1
    cap = jnp.floor((cnt + (_BLK - 1)) / _BLK) * _BLK
    ei = lax.broadcasted_iota(jnp.int32, (_E, _E), 0)
    ej = lax.broadcasted_iota(jnp.int32, (_E, _E), 1)
    ltm = (ei < ej).astype(jnp.float32)
    off = lax.dot_general(cap, ltm, (((1,), (0,)), ((), ())),
                          preferred_element_type=jnp.float32)
    dst1_ref[...] = (jnp.sum(off * oh1, axis=1, keepdims=True)
                     + rank1).astype(jnp.int32)
    dst2_ref[...] = (jnp.sum(off * oh2, axis=1, keepdims=True)
                     + rank2).astype(jnp.int32)
    s1_ref[...] = s1
    s2_ref[...] = 1.0 - s1
    cnt_ref[...] = cnt.astype(jnp.int32)


def _gate_call(x, w_gate):
    return pl.pallas_call(
        _gate_body,
        out_shape=(
            jax.ShapeDtypeStruct((_T, 1), jnp.int32),
            jax.ShapeDtypeStruct((_T, 1), jnp.int32),
            jax.ShapeDtypeStruct((_T, 1), jnp.float32),
            jax.ShapeDtypeStruct((_T, 1), jnp.float32),
            jax.ShapeDtypeStruct((1, _E), jnp.int32),
            jax.ShapeDtypeStruct((_T, _D // 2), jnp.int32),
        ),
    )(x, w_gate)


def _mesh():
    return plsc.VectorSubcoreMesh(core_axis_name="c", subcore_axis_name="s",
                                  num_cores=_NC, num_subcores=_NS)


@functools.cache
def _dispatch_call():
    return functools.partial(
        pl.kernel,
        out_type=(jax.ShapeDtypeStruct((_NPAD, _D // 2), jnp.int32),
                  jax.ShapeDtypeStruct((_NPAD,), jnp.float32)),
        mesh=_mesh(),
        scratch_types=[
            pltpu.VMEM((_NPAD,), jnp.int32),
            pltpu.VMEM((_NPAD,), jnp.float32),
            pltpu.VMEM((_T,), jnp.int32),
            pltpu.VMEM((_T,), jnp.int32),
            pltpu.VMEM((_T,), jnp.float32),
            pltpu.VMEM((_T,), jnp.float32),
            pltpu.VMEM((_GCH, _D // 2), jnp.int32),
            pltpu.VMEM((_GCH, _D // 2), jnp.int32),
            pltpu.VMEM((_NPAD // _NW,), jnp.int32),
            pltpu.SemaphoreType.DMA,
            pltpu.SemaphoreType.DMA,
        ],
        compiler_params=pltpu.CompilerParams(needs_layout_passes=False),
    )(_dispatch)


def _dispatch(xb_hbm, dst1_hbm, dst2_hbm, s1_hbm, s2_hbm, xs_hbm, ws_hbm,
              tok_loc, w_loc, dst1_loc, dst2_loc, s1_loc, s2_loc,
              rows0, rows1, idxbuf, sem0, sem1):
    wid = lax.axis_index("s") * _NC + lax.axis_index("c")
    with jax.named_scope("disp_stage"):
        pltpu.sync_copy(dst1_hbm, dst1_loc)
        pltpu.sync_copy(dst2_hbm, dst2_loc)
        pltpu.sync_copy(s1_hbm, s1_loc)
        pltpu.sync_copy(s2_hbm, s2_loc)

    def _zero(i, _):
        tok_loc[pl.ds(i * 16, 16)] = jnp.bitwise_and(
            lax.iota(jnp.int32, 16) + i * 16, _T - 1)
        w_loc[pl.ds(i * 16, 16)] = jnp.zeros((16,), jnp.float32)
        return ()
    with jax.named_scope("disp_zero"):
        lax.fori_loop(0, _NPAD // 16, _zero, (), unroll=4)

    def _scatter1(i, _):
        sl = pl.ds(i * 16, 16)
        tv = lax.iota(jnp.int32, 16) + i * 16
        plsc.store_scatter(tok_loc, [dst1_loc[sl]], tv)
        plsc.store_scatter(w_loc, [dst1_loc[sl]], s1_loc[sl])
        return ()
    with jax.named_scope("disp_scat1"):
        lax.fori_loop(0, _T // 16, _scatter1, (), unroll=4)

    def _scatter2(i, _):
        sl = pl.ds(i * 16, 16)
        tv = lax.iota(jnp.int32, 16) + i * 16
        plsc.store_scatter(tok_loc, [dst2_loc[sl]], tv)
        plsc.store_scatter(w_loc, [dst2_loc[sl]], s2_loc[sl])
        return ()
    with jax.named_scope("disp_scat2"):
        lax.fori_loop(0, _T // 16, _scatter2, (), unroll=4)

    span = _NPAD // _NW
    base = wid * span
    with jax.named_scope("disp_ws"):
        pltpu.sync_copy(w_loc.at[pl.ds(base, span)],
                        ws_hbm.at[pl.ds(base, span)])

    def _cpidx(i, _):
        idxbuf[pl.ds(i * 16, 16)] = tok_loc[pl.ds(base + i * 16, 16)]
        return ()
    with jax.named_scope("disp_cpidx"):
        lax.fori_loop(0, span // 16, _cpidx, (), unroll=4)

    with jax.named_scope("disp_gather"):
        nch = span // _GCH
        rows = (rows0, rows1)
        sems = (sem0, sem1)
        cps = [None, None]
        cps[0] = pltpu.async_copy(
            xb_hbm.at[idxbuf.at[pl.ds(0, _GCH)]], rows0, sem0)
        for c in range(nch):
            if c + 1 < nch:
                cps[(c + 1) % 2] = pltpu.async_copy(
                    xb_hbm.at[idxbuf.at[pl.ds((c + 1) * _GCH, _GCH)]],
                    rows[(c + 1) % 2], sems[(c + 1) % 2])
            cps[c % 2].wait()
            pltpu.sync_copy(rows[c % 2],
                            xs_hbm.at[pl.ds(base + c * _GCH, _GCH)])


def _ffn_body(sched_ref, xs_ref, w1_ref, w2_ref, w3_ref, ws_ref, ys_ref):
    j = pl.program_id(0)

    @pl.when(sched_ref[2, j] == 1)
    def _():
        w = xs_ref[...]
        xlo = lax.bitcast_convert_type(
            lax.shift_left(w, 16), jnp.float32).astype(jnp.bfloat16)
        xhi = lax.bitcast_convert_type(
            jnp.bitwise_and(w, jnp.int32(-65536)),
            jnp.float32).astype(jnp.bfloat16)
        w1 = w1_ref[0].astype(jnp.bfloat16)
        w3 = w3_ref[0].astype(jnp.bfloat16)
        w2 = w2_ref[0].astype(jnp.bfloat16)
        dn = (((1,), (1,)), ((), ()))
        a = (lax.dot_general(xlo, w1[:, :_D // 2], dn,
                             preferred_element_type=jnp.float32)
             + lax.dot_general(xhi, w1[:, _D // 2:], dn,
                               preferred_element_type=jnp.float32))
        b = (lax.dot_general(xlo, w3[:, :_D // 2], dn,
                             preferred_element_type=jnp.float32)
             + lax.dot_general(xhi, w3[:, _D // 2:], dn,
                               preferred_element_type=jnp.float32))
        h = (a * lax.logistic(a) * b).astype(jnp.bfloat16)
        oe = lax.dot_general(h, w2, (((1,), (0,)), ((), ())),
                             preferred_element_type=jnp.float32)
        ys_ref[...] = oe * ws_ref[...]


def _ffn_call(sched, xs, w1, w2, w3, ws):
    grid_spec = pltpu.PrefetchScalarGridSpec(
        num_scalar_prefetch=1,
        grid=(_NT,),
        in_specs=[
            pl.BlockSpec((_BLK, _D // 2), lambda j, s: (s[0, j], 0)),
            pl.BlockSpec((1, _F, _D), lambda j, s: (s[1, j], 0, 0)),
            pl.BlockSpec((1, _F, _D), lambda j, s: (s[1, j], 0, 0)),
            pl.BlockSpec((1, _F, _D), lambda j, s: (s[1, j], 0, 0)),
            pl.BlockSpec((_BLK, 1), lambda j, s: (s[0, j], 0)),
        ],
        out_specs=pl.BlockSpec((_BLK, _D), lambda j, s: (s[0, j], 0)),
    )
    return pl.pallas_call(
        _ffn_body,
        grid_spec=grid_spec,
        out_shape=jax.ShapeDtypeStruct((_NPAD, _D), jnp.float32),
        compiler_params=pltpu.CompilerParams(
            dimension_semantics=("arbitrary",)),
    )(sched, xs, w1, w2, w3, ws)


@functools.cache
def _combine_call():
    return functools.partial(
        pl.kernel,
        out_type=jax.ShapeDtypeStruct((_T, _D), jnp.float32),
        mesh=_mesh(),
        scratch_types=[
            pltpu.VMEM((_T // _NW,), jnp.int32),
            pltpu.VMEM((_T // _NW,), jnp.int32),
            pltpu.VMEM((_CCH, _D), jnp.float32),
            pltpu.VMEM((_CCH, _D), jnp.float32),
            pltpu.SemaphoreType.DMA,
            pltpu.SemaphoreType.DMA,
        ],
        compiler_params=pltpu.CompilerParams(needs_layout_passes=False),
    )(_combine)


def _combine(ys_hbm, dst1_hbm, dst2_hbm, out_hbm,
             idx1, idx2, buf1, buf2, sem1, sem2):
    wid = lax.axis_index("s") * _NC + lax.axis_index("c")
    span = _T // _NW
    base = wid * span
    pltpu.sync_copy(dst1_hbm.at[pl.ds(base, span)], idx1)
    pltpu.sync_copy(dst2_hbm.at[pl.ds(base, span)], idx2)
    for c in range(span // _CCH):
        a1 = pltpu.async_copy(ys_hbm.at[idx1.at[pl.ds(c * _CCH, _CCH)]],
                              buf1, sem1)
        a2 = pltpu.async_copy(ys_hbm.at[idx2.at[pl.ds(c * _CCH, _CCH)]],
                              buf2, sem2)
        a1.wait()
        a2.wait()

        def _add(r, _):
            for cc in range(_D // 16):
                sl = pl.ds(cc * 16, 16)
                buf1[r, sl] = buf1[r, sl] + buf2[r, sl]
            return ()
        lax.fori_loop(0, _CCH, _add, ())
        pltpu.sync_copy(buf1, out_hbm.at[pl.ds(base + c * _CCH, _CCH)])


def kernel(x, w_gate, w1, w2, w3):
    dst1, dst2, s1, s2, cnt, xb_i = _gate_call(x, w_gate)
    d1 = dst1.reshape(_T)
    d2 = dst2.reshape(_T)
    cnt = cnt.reshape(_E)
    tiles = (cnt + _BLK - 1) // _BLK
    at = jnp.sum(tiles)
    jr = jnp.arange(_NT, dtype=jnp.int32)
    jcl = jnp.minimum(jr, at - 1)
    exps = jnp.searchsorted(jnp.cumsum(tiles), jcl, side="right")
    sched = jnp.stack([jcl, exps.astype(jnp.int32),
                       (jr < at).astype(jnp.int32)])
    xs_i, ws = _dispatch_call()(xb_i, d1, d2, s1.reshape(_T), s2.reshape(_T))
    ys = _ffn_call(sched, xs_i, w1, w2, w3, ws.reshape(_NPAD, 1))
    return _combine_call()(ys, d1, d2)


# speedup vs baseline: 1.7478x; 1.0105x over previous
import functools

import jax
import jax.numpy as jnp
from jax import lax
from jax.experimental import pallas as pl
from jax.experimental.pallas import tpu as pltpu
from jax.experimental.pallas import tpu_sc as plsc

_E = 8
_K = 2
_T = 2048
_D = 1024
_F = 512
_BLK = 256
_NT = (_T * _K) // _BLK + _E - 1
_NPAD = 6144
_NC, _NS = 2, 16
_NW = _NC * _NS
_GCH = 32
_CCH = 32


def _cumsum0(a, n):
    k = 1
    while k < n:
        a = a + jnp.concatenate(
            [jnp.zeros((k, a.shape[1]), a.dtype), a[:-k]], axis=0)
        k *= 2
    return a


def _bf16_bits(xf):
    u = lax.bitcast_convert_type(xf, jnp.int32)
    return u + 0x7FFF + jnp.bitwise_and(lax.shift_right_logical(u, 16), 1)


def _gate_body(x_ref, wg_ref, dst1_ref, dst2_ref, s1_ref, s2_ref, sched_ref,
               xb_ref):
    x = x_ref[...]
    rl = _bf16_bits(x[:, :_D // 2])
    rh = _bf16_bits(x[:, _D // 2:])
    xb_ref[...] = jnp.bitwise_or(
        jnp.bitwise_and(rh, jnp.int32(-65536)),
        jnp.bitwise_and(lax.shift_right_logical(rl, 16), 0xFFFF))
    logits = lax.dot_general(x, wg_ref[...], (((1,), (1,)), ((), ())),
                             preferred_element_type=jnp.float32)
    cols = lax.broadcasted_iota(jnp.int32, logits.shape, 1)
    m1 = jnp.max(logits, axis=1, keepdims=True)
    i1 = jnp.min(jnp.where(logits == m1, cols, _E), axis=1, keepdims=True)
    oh1 = (cols == i1).astype(jnp.float32)
    rest = jnp.where(cols == i1, -jnp.inf, logits)
    m2 = jnp.max(rest, axis=1, keepdims=True)
    i2 = jnp.min(jnp.where(rest == m2, cols, _E), axis=1, keepdims=True)
    oh2 = (cols == i2).astype(jnp.float32)
    s1 = 1.0 / (1.0 + jnp.exp(m2 - m1))

    c0 = _cumsum0(oh1, _T)
    tot0 = c0[_T - 1:_T, :]
    c1 = _cumsum0(oh2, _T)
    tot1 = c1[_T - 1:_T, :]
    rank1 = jnp.sum((c0 - oh1) * oh1, axis=1, keepdims=True)
    rank2 = jnp.sum((c1 - oh2 + tot0) * oh2, axis=1, keepdims=True)
    cnt = tot0 + tot1
    cap = jnp.floor((cnt + (_BLK - 1)) / _BLK) * _BLK
    ei = lax.broadcasted_iota(jnp.int32, (_E, _E), 0)
    ej = lax.broadcasted_iota(jnp.int32, (_E, _E), 1)
    ltm = (ei < ej).astype(jnp.float32)
    off = lax.dot_general(cap, ltm, (((1,), (0,)), ((), ())),
                          preferred_element_type=jnp.float32)
    dst1_ref[...] = (jnp.sum(off * oh1, axis=1, keepdims=True)
                     + rank1).astype(jnp.int32)
    dst2_ref[...] = (jnp.sum(off * oh2, axis=1, keepdims=True)
                     + rank2).astype(jnp.int32)
    s1_ref[...] = s1
    s2_ref[...] = 1.0 - s1
    tiles = cap * (1.0 / _BLK)
    cum = lax.dot_general(tiles, (ei <= ej).astype(jnp.float32),
                          (((1,), (0,)), ((), ())),
                          preferred_element_type=jnp.float32)
    at = jnp.sum(tiles, axis=1, keepdims=True).astype(jnp.int32)
    jr = lax.broadcasted_iota(jnp.int32, (1, _NT), 1)
    jcl = jnp.minimum(jr, at - 1)
    exps = jnp.zeros((1, _NT), jnp.int32)
    for e in range(_E):
        exps = exps + (cum[:, e:e + 1].astype(jnp.int32) <= jcl)
    sched_ref[0:1, :] = jcl
    sched_ref[1:2, :] = exps
    sched_ref[2:3, :] = (jr < at).astype(jnp.int32)


def _gate_call(x, w_gate):
    return pl.pallas_call(
        _gate_body,
        out_shape=(
            jax.ShapeDtypeStruct((_T, 1), jnp.int32),
            jax.ShapeDtypeStruct((_T, 1), jnp.int32),
            jax.ShapeDtypeStruct((_T, 1), jnp.float32),
            jax.ShapeDtypeStruct((_T, 1), jnp.float32),
            jax.ShapeDtypeStruct((3, _NT), jnp.int32),
            jax.ShapeDtypeStruct((_T, _D // 2), jnp.int32),
        ),
    )(x, w_gate)


def _mesh():
    return plsc.VectorSubcoreMesh(core_axis_name="c", subcore_axis_name="s",
                                  num_cores=_NC, num_subcores=_NS)


@functools.cache
def _dispatch_call():
    return functools.partial(
        pl.kernel,
        out_type=(jax.ShapeDtypeStruct((_NPAD, _D // 2), jnp.int32),
                  jax.ShapeDtypeStruct((_NPAD,), jnp.float32)),
        mesh=_mesh(),
        scratch_types=[
            pltpu.VMEM((_NPAD,), jnp.int32),
            pltpu.VMEM((_NPAD,), jnp.float32),
            pltpu.VMEM((_T,), jnp.int32),
            pltpu.VMEM((_T,), jnp.int32),
            pltpu.VMEM((_T,), jnp.float32),
            pltpu.VMEM((_T,), jnp.float32),
            pltpu.VMEM((_GCH, _D // 2), jnp.int32),
            pltpu.VMEM((_GCH, _D // 2), jnp.int32),
            pltpu.SemaphoreType.DMA,
            pltpu.SemaphoreType.DMA,
        ],
        compiler_params=pltpu.CompilerParams(needs_layout_passes=False),
    )(_dispatch)


def _dispatch(xb_hbm, dst1_hbm, dst2_hbm, s1_hbm, s2_hbm, xs_hbm, ws_hbm,
              tok_loc, w_loc, dst1_loc, dst2_loc, s1_loc, s2_loc,
              rows0, rows1, sem0, sem1):
    wid = lax.axis_index("s") * _NC + lax.axis_index("c")
    pltpu.sync_copy(dst1_hbm, dst1_loc)
    pltpu.sync_copy(dst2_hbm, dst2_loc)
    pltpu.sync_copy(s1_hbm, s1_loc)
    pltpu.sync_copy(s2_hbm, s2_loc)

    def _zero(i, _):
        tok_loc[pl.ds(i * 16, 16)] = jnp.bitwise_and(
            lax.iota(jnp.int32, 16) + i * 16, _T - 1)
        w_loc[pl.ds(i * 16, 16)] = jnp.zeros((16,), jnp.float32)
        return ()
    lax.fori_loop(0, _NPAD // 16, _zero, (), unroll=4)

    def _scatter1(i, _):
        sl = pl.ds(i * 16, 16)
        tv = lax.iota(jnp.int32, 16) + i * 16
        plsc.store_scatter(tok_loc, [dst1_loc[sl]], tv)
        plsc.store_scatter(w_loc, [dst1_loc[sl]], s1_loc[sl])
        return ()
    lax.fori_loop(0, _T // 16, _scatter1, (), unroll=4)

    def _scatter2(i, _):
        sl = pl.ds(i * 16, 16)
        tv = lax.iota(jnp.int32, 16) + i * 16
        plsc.store_scatter(tok_loc, [dst2_loc[sl]], tv)
        plsc.store_scatter(w_loc, [dst2_loc[sl]], s2_loc[sl])
        return ()
    lax.fori_loop(0, _T // 16, _scatter2, (), unroll=4)

    span = _NPAD // _NW
    base = wid * span
    pltpu.sync_copy(w_loc.at[pl.ds(base, span)],
                    ws_hbm.at[pl.ds(base, span)])

    nch = span // _GCH
    rows = (rows0, rows1)
    sems = (sem0, sem1)
    cps = [None, None]
    cps[0] = pltpu.async_copy(
        xb_hbm.at[tok_loc.at[pl.ds(base, _GCH)]], rows0, sem0)
    for c in range(nch):
        if c + 1 < nch:
            cps[(c + 1) % 2] = pltpu.async_copy(
                xb_hbm.at[tok_loc.at[pl.ds(base + (c + 1) * _GCH, _GCH)]],
                rows[(c + 1) % 2], sems[(c + 1) % 2])
        cps[c % 2].wait()
        pltpu.sync_copy(rows[c % 2],
                        xs_hbm.at[pl.ds(base + c * _GCH, _GCH)])


def _ffn_body(sched_ref, xs_ref, w1_ref, w2_ref, w3_ref, ws_ref, ys_ref):
    j = pl.program_id(0)

    @pl.when(sched_ref[2, j] == 1)
    def _():
        w = xs_ref[...]
        xlo = lax.bitcast_convert_type(
            lax.shift_left(w, 16), jnp.float32).astype(jnp.bfloat16)
        xhi = lax.bitcast_convert_type(
            jnp.bitwise_and(w, jnp.int32(-65536)),
            jnp.float32).astype(jnp.bfloat16)
        w1 = w1_ref[0].astype(jnp.bfloat16)
        w3 = w3_ref[0].astype(jnp.bfloat16)
        w2 = w2_ref[0].astype(jnp.bfloat16)
        dn = (((1,), (1,)), ((), ()))
        a = (lax.dot_general(xlo, w1[:, :_D // 2], dn,
                             preferred_element_type=jnp.float32)
             + lax.dot_general(xhi, w1[:, _D // 2:], dn,
                               preferred_element_type=jnp.float32))
        b = (lax.dot_general(xlo, w3[:, :_D // 2], dn,
                             preferred_element_type=jnp.float32)
             + lax.dot_general(xhi, w3[:, _D // 2:], dn,
                               preferred_element_type=jnp.float32))
        h = (a * lax.logistic(a) * b).astype(jnp.bfloat16)
        oe = lax.dot_general(h, w2, (((1,), (0,)), ((), ())),
                             preferred_element_type=jnp.float32)
        y = oe * ws_ref[...]
        rl = _bf16_bits(y[:, :_D // 2])
        rh = _bf16_bits(y[:, _D // 2:])
        ys_ref[...] = jnp.bitwise_or(
            jnp.bitwise_and(rh, jnp.int32(-65536)),
            jnp.bitwise_and(lax.shift_right_logical(rl, 16), 0xFFFF))


def _ffn_call(sched, xs, w1, w2, w3, ws):
    grid_spec = pltpu.PrefetchScalarGridSpec(
        num_scalar_prefetch=1,
        grid=(_NT,),
        in_specs=[
            pl.BlockSpec((_BLK, _D // 2), lambda j, s: (s[0, j], 0)),
            pl.BlockSpec((1, _F, _D), lambda j, s: (s[1, j], 0, 0)),
            pl.BlockSpec((1, _F, _D), lambda j, s: (s[1, j], 0, 0)),
            pl.BlockSpec((1, _F, _D), lambda j, s: (s[1, j], 0, 0)),
            pl.BlockSpec((_BLK, 1), lambda j, s: (s[0, j], 0)),
        ],
        out_specs=pl.BlockSpec((_BLK, _D // 2), lambda j, s: (s[0, j], 0)),
    )
    return pl.pallas_call(
        _ffn_body,
        grid_spec=grid_spec,
        out_shape=jax.ShapeDtypeStruct((_NPAD, _D // 2), jnp.int32),
        compiler_params=pltpu.CompilerParams(
            dimension_semantics=("arbitrary",)),
    )(sched, xs, w1, w2, w3, ws)


@functools.cache
def _combine_call():
    return functools.partial(
        pl.kernel,
        out_type=jax.ShapeDtypeStruct((_T, _D), jnp.float32),
        mesh=_mesh(),
        scratch_types=[
            pltpu.VMEM((_T // _NW,), jnp.int32),
            pltpu.VMEM((_T // _NW,), jnp.int32),
            pltpu.VMEM((_CCH, _D // 2), jnp.int32),
            pltpu.VMEM((_CCH, _D // 2), jnp.int32),
            pltpu.VMEM((_CCH, _D), jnp.float32),
            pltpu.SemaphoreType.DMA,
            pltpu.SemaphoreType.DMA,
        ],
        compiler_params=pltpu.CompilerParams(needs_layout_passes=False),
    )(_combine)


def _combine(ys_hbm, dst1_hbm, dst2_hbm, out_hbm,
             idx1, idx2, buf1, buf2, obuf, sem1, sem2):
    wid = lax.axis_index("s") * _NC + lax.axis_index("c")
    span = _T // _NW
    base = wid * span
    pltpu.sync_copy(dst1_hbm.at[pl.ds(base, span)], idx1)
    pltpu.sync_copy(dst2_hbm.at[pl.ds(base, span)], idx2)
    himask = jnp.full((16,), -65536, jnp.int32)
    for c in range(span // _CCH):
        a1 = pltpu.async_copy(ys_hbm.at[idx1.at[pl.ds(c * _CCH, _CCH)]],
                              buf1, sem1)
        a2 = pltpu.async_copy(ys_hbm.at[idx2.at[pl.ds(c * _CCH, _CCH)]],
                              buf2, sem2)
        a1.wait()
        a2.wait()

        def _add(r, _):
            for cc in range(_D // 32):
                sl = pl.ds(cc * 16, 16)
                w1v = buf1[r, sl]
                w2v = buf2[r, sl]
                lo = (lax.bitcast_convert_type(
                          lax.shift_left(w1v, 16), jnp.float32)
                      + lax.bitcast_convert_type(
                          lax.shift_left(w2v, 16), jnp.float32))
                hi = (lax.bitcast_convert_type(
                          jnp.bitwise_and(w1v, himask), jnp.float32)
                      + lax.bitcast_convert_type(
                          jnp.bitwise_and(w2v, himask), jnp.float32))
                obuf[r, sl] = lo
                obuf[r, pl.ds(_D // 2 + cc * 16, 16)] = hi
            return ()
        lax.fori_loop(0, _CCH, _add, ())
        pltpu.sync_copy(obuf, out_hbm.at[pl.ds(base + c * _CCH, _CCH)])


def kernel(x, w_gate, w1, w2, w3):
    dst1, dst2, s1, s2, sched, xb_i = _gate_call(x, w_gate)
    d1 = dst1.reshape(_T)
    d2 = dst2.reshape(_T)
    xs_i, ws = _dispatch_call()(xb_i, d1, d2, s1.reshape(_T), s2.reshape(_T))
    ys = _ffn_call(sched, xs_i, w1, w2, w3, ws.reshape(_NPAD, 1))
    return _combine_call()(ys, d1, d2)
